# baseline (speedup 1.0000x reference)
import functools

import jax
import jax.numpy as jnp
from jax import lax
from jax.experimental import pallas as pl
from jax.experimental.pallas import tpu as pltpu
from jax.experimental.pallas import tpu_sc as plsc

VOCAB = 8192
HIDDEN = 256
GEN_HIDDEN = 4096
NUM_TOKENS = 8192
HALF = HIDDEN // 2

NUM_CHUNKS = 2
CHUNK = NUM_TOKENS // NUM_CHUNKS

NUM_WORKERS = 32
TOK_PER_W = CHUNK // NUM_WORKERS

TILE_M = 1024
GRID_M = CHUNK // TILE_M

PACK_TILE = 2048
PACK_GRID = VOCAB // PACK_TILE


def _sc_gather_body(ids_hbm, table_hbm, out_hbm, idx_v, rows_v, sem):
    wid = lax.axis_index("s") * 2 + lax.axis_index("c")
    base = wid * TOK_PER_W
    pltpu.sync_copy(ids_hbm.at[pl.ds(base, TOK_PER_W)], idx_v)
    pltpu.async_copy(table_hbm.at[idx_v], rows_v, sem).wait()
    pltpu.sync_copy(rows_v, out_hbm.at[pl.ds(base, TOK_PER_W)])


def _sc_gather(ids_chunk, table_i32):
    mesh = plsc.VectorSubcoreMesh(core_axis_name="c", subcore_axis_name="s")
    fn = pl.kernel(
        _sc_gather_body,
        out_type=jax.ShapeDtypeStruct((CHUNK, HALF), jnp.int32),
        mesh=mesh,
        scratch_types=[
            pltpu.VMEM((TOK_PER_W,), jnp.int32),
            pltpu.VMEM((TOK_PER_W, HALF), jnp.int32),
            pltpu.SemaphoreType.DMA,
        ],
    )
    return fn(ids_chunk, table_i32)


def _pack_body(e_ref, out_ref):
    f32 = jnp.float32
    e = e_ref[...]
    lo = pltpu.bitcast(e[:, :HALF].astype(f32), jnp.uint32) >> 16
    hi = pltpu.bitcast(e[:, HALF:].astype(f32), jnp.uint32) & jnp.uint32(
        0xFFFF0000
    )
    out_ref[...] = pltpu.bitcast(lo | hi, jnp.int32)


def _pack_table(embedding):
    return pl.pallas_call(
        _pack_body,
        grid=(PACK_GRID,),
        in_specs=[pl.BlockSpec((PACK_TILE, HIDDEN), lambda i: (i, 0))],
        out_specs=pl.BlockSpec((PACK_TILE, HALF), lambda i: (i, 0)),
        out_shape=jax.ShapeDtypeStruct((VOCAB, HALF), jnp.int32),
    )(embedding)


def _tc_mse_body(finalize, x_ref, w_ref, b_ref, l_ref, p_ref, out_ref):
    i = pl.program_id(0)
    f32 = jnp.float32
    bf16 = jnp.bfloat16
    acc = lax.dot_general(
        x_ref[...], w_ref[...],
        dimension_numbers=(((1,), (1,)), ((), ())),
        preferred_element_type=f32,
    ).astype(bf16)
    feat = (acc.astype(f32) + b_ref[...].astype(f32)).astype(bf16)
    words = l_ref[...]
    lab_lo = pltpu.bitcast(words << 16, f32)
    lab_hi = pltpu.bitcast(words & jnp.int32(-65536), f32)
    f_lo = feat[:, :HALF].astype(f32)
    f_hi = feat[:, HALF:].astype(f32)
    d_lo = (f_lo - lab_lo).astype(bf16).astype(f32)
    d_hi = (f_hi - lab_hi).astype(bf16).astype(f32)
    part = jnp.sum((d_lo * d_lo).astype(bf16).astype(f32)) + jnp.sum(
        (d_hi * d_hi).astype(bf16).astype(f32)
    )

    @pl.when(i == 0)
    def _():
        out_ref[0, 0] = p_ref[0, 0]

    out_ref[0, 0] += part

    if finalize:
        @pl.when(i == GRID_M - 1)
        def _():
            out_ref[0, 0] = out_ref[0, 0] * (1.0 / (NUM_TOKENS * HIDDEN))


def _tc_mse(x_chunk, w, b2d, labels_i32, partial, finalize):
    return pl.pallas_call(
        functools.partial(_tc_mse_body, finalize),
        grid=(GRID_M,),
        in_specs=[
            pl.BlockSpec((TILE_M, GEN_HIDDEN), lambda i: (i, 0)),
            pl.BlockSpec((HIDDEN, GEN_HIDDEN), lambda i: (0, 0)),
            pl.BlockSpec((1, HIDDEN), lambda i: (0, 0)),
            pl.BlockSpec((TILE_M, HALF), lambda i: (i, 0)),
            pl.BlockSpec(memory_space=pltpu.SMEM),
        ],
        out_specs=pl.BlockSpec(memory_space=pltpu.SMEM),
        out_shape=jax.ShapeDtypeStruct((1, 1), jnp.float32),
    )(x_chunk, w, b2d, labels_i32, partial)


def kernel(input_ids, generated_hidden_states, embedding, W, b):
    ids_flat = input_ids.reshape(NUM_TOKENS).astype(jnp.int32) - 4
    table_i32 = _pack_table(embedding)
    x = generated_hidden_states.reshape(NUM_TOKENS, GEN_HIDDEN)
    b2d = b.reshape(1, HIDDEN)
    partial = jnp.zeros((1, 1), jnp.float32)
    for c in range(NUM_CHUNKS):
        sl = slice(c * CHUNK, (c + 1) * CHUNK)
        labels_c = _sc_gather(ids_flat[sl], table_i32)
        partial = _tc_mse(
            x[sl], W, b2d, labels_c, partial, finalize=(c == NUM_CHUNKS - 1)
        )
    return partial.reshape(()).astype(jnp.bfloat16)


# speedup vs baseline: 1.7821x; 1.7821x over previous
import jax
import jax.numpy as jnp
from jax import lax
from jax.experimental import pallas as pl
from jax.experimental.pallas import tpu as pltpu
from jax.experimental.pallas import tpu_sc as plsc

VOCAB = 8192
HIDDEN = 256
GEN_HIDDEN = 4096
NUM_TOKENS = 8192
HALF = HIDDEN // 2

NUM_WORKERS = 32
TOK_PER_W = NUM_TOKENS // NUM_WORKERS
IDX_CHUNK = 128

TILE_M = 1024
GRID_M = NUM_TOKENS // TILE_M

PACK_TILE = 2048
PACK_GRID = VOCAB // PACK_TILE


def _sc_gather_body(ids_hbm, table_hbm, out_hbm, idx_v, rows_v, sem):
    wid = lax.axis_index("s") * 2 + lax.axis_index("c")
    base = wid * TOK_PER_W
    pltpu.sync_copy(ids_hbm.at[pl.ds(base, TOK_PER_W)], idx_v)
    copies = []
    for j in range(TOK_PER_W // IDX_CHUNK):
        sl = pl.ds(j * IDX_CHUNK, IDX_CHUNK)
        copies.append(
            pltpu.async_copy(table_hbm.at[idx_v.at[sl]], rows_v.at[sl], sem)
        )
    for c in copies:
        c.wait()
    pltpu.sync_copy(rows_v, out_hbm.at[pl.ds(base, TOK_PER_W)])


def _sc_gather(ids_flat, table_i32):
    mesh = plsc.VectorSubcoreMesh(core_axis_name="c", subcore_axis_name="s")
    fn = pl.kernel(
        _sc_gather_body,
        out_type=jax.ShapeDtypeStruct((NUM_TOKENS, HALF), jnp.int32),
        mesh=mesh,
        scratch_types=[
            pltpu.VMEM((TOK_PER_W,), jnp.int32),
            pltpu.VMEM((TOK_PER_W, HALF), jnp.int32),
            pltpu.SemaphoreType.DMA,
        ],
    )
    return fn(ids_flat, table_i32)


def _pack_body(e_ref, out_ref):
    f32 = jnp.float32
    e = e_ref[...]
    lo = pltpu.bitcast(e[:, :HALF].astype(f32), jnp.uint32) >> 16
    hi = pltpu.bitcast(e[:, HALF:].astype(f32), jnp.uint32) & jnp.uint32(
        0xFFFF0000
    )
    out_ref[...] = pltpu.bitcast(lo | hi, jnp.int32)


def _pack_table(embedding):
    return pl.pallas_call(
        _pack_body,
        grid=(PACK_GRID,),
        in_specs=[pl.BlockSpec((PACK_TILE, HIDDEN), lambda i: (i, 0))],
        out_specs=pl.BlockSpec((PACK_TILE, HALF), lambda i: (i, 0)),
        out_shape=jax.ShapeDtypeStruct((VOCAB, HALF), jnp.int32),
    )(embedding)


def _tc_mse_body(x_ref, w_ref, b_ref, l_ref, out_ref):
    i = pl.program_id(0)
    f32 = jnp.float32
    bf16 = jnp.bfloat16
    acc = lax.dot_general(
        x_ref[...], w_ref[...],
        dimension_numbers=(((1,), (1,)), ((), ())),
        preferred_element_type=f32,
    ).astype(bf16)
    feat = (acc.astype(f32) + b_ref[...].astype(f32)).astype(bf16)
    words = l_ref[...]
    lab_lo = pltpu.bitcast(words << 16, f32)
    lab_hi = pltpu.bitcast(words & jnp.int32(-65536), f32)
    f_lo = feat[:, :HALF].astype(f32)
    f_hi = feat[:, HALF:].astype(f32)
    d_lo = (f_lo - lab_lo).astype(bf16).astype(f32)
    d_hi = (f_hi - lab_hi).astype(bf16).astype(f32)
    part = jnp.sum((d_lo * d_lo).astype(bf16).astype(f32)) + jnp.sum(
        (d_hi * d_hi).astype(bf16).astype(f32)
    )

    @pl.when(i == 0)
    def _():
        out_ref[0, 0] = 0.0

    out_ref[0, 0] += part

    @pl.when(i == GRID_M - 1)
    def _():
        out_ref[0, 0] = out_ref[0, 0] * (1.0 / (NUM_TOKENS * HIDDEN))


def _tc_mse(x, w, b2d, labels_i32):
    return pl.pallas_call(
        _tc_mse_body,
        grid=(GRID_M,),
        in_specs=[
            pl.BlockSpec((TILE_M, GEN_HIDDEN), lambda i: (i, 0)),
            pl.BlockSpec((HIDDEN, GEN_HIDDEN), lambda i: (0, 0)),
            pl.BlockSpec((1, HIDDEN), lambda i: (0, 0)),
            pl.BlockSpec((TILE_M, HALF), lambda i: (i, 0)),
        ],
        out_specs=pl.BlockSpec(memory_space=pltpu.SMEM),
        out_shape=jax.ShapeDtypeStruct((1, 1), jnp.float32),
    )(x, w, b2d, labels_i32)


def kernel(input_ids, generated_hidden_states, embedding, W, b):
    ids_flat = input_ids.reshape(NUM_TOKENS).astype(jnp.int32) - 4
    table_i32 = _pack_table(embedding)
    labels_i32 = _sc_gather(ids_flat, table_i32)
    x = generated_hidden_states.reshape(NUM_TOKENS, GEN_HIDDEN)
    out = _tc_mse(x, W, b.reshape(1, HIDDEN), labels_i32)
    return out.reshape(()).astype(jnp.bfloat16)
